# baseline (speedup 1.0000x reference)
import jax
import jax.numpy as jnp
from jax.experimental import pallas as pl
from jax.experimental.pallas import tpu as pltpu

_BLK = 16384
_NSPLIT = 8
_SUB = _BLK // _NSPLIT
_NOUT = 8


def _mlp_kernel(*refs):
    x_refs = refs[:_NSPLIT]
    w1_ref, b1_ref, w2_ref, b2_ref, w3_ref, o_ref = refs[_NSPLIT:]
    w1 = w1_ref[...].astype(jnp.bfloat16)
    w2 = w2_ref[...].astype(jnp.bfloat16)
    b1 = b1_ref[...].astype(jnp.bfloat16)
    b2 = b2_ref[...].astype(jnp.bfloat16)
    w3 = jnp.broadcast_to(w3_ref[...] * 0.125, (_NOUT, 64)).astype(jnp.bfloat16)
    for s, x_ref in enumerate(x_refs):
        x = x_ref[...].astype(jnp.bfloat16)
        h = jax.lax.dot_general(
            x, w1, (((1,), (1,)), ((), ())),
            preferred_element_type=jnp.float32)
        h = jnp.maximum(h.astype(jnp.bfloat16) + b1, jnp.bfloat16(0))
        h = jax.lax.dot_general(
            h, w2, (((1,), (1,)), ((), ())),
            preferred_element_type=jnp.float32)
        h = jnp.maximum(h.astype(jnp.bfloat16) + b2, jnp.bfloat16(0))
        o_ref[:, pl.ds(s * _SUB, _SUB)] = jax.lax.dot_general(
            w3, h, (((1,), (1,)), ((), ())),
            preferred_element_type=jnp.float32)


def kernel(input, W1, b1, W2, b2, W3, b3):
    B, D = input.shape
    grid = (B // _BLK,)
    b1r = b1.reshape(1, 64)
    b2r = b2.reshape(1, 64)
    x_specs = [
        pl.BlockSpec((_SUB, D), lambda i, s=s: (i * _NSPLIT + s, 0))
        for s in range(_NSPLIT)
    ]
    out = pl.pallas_call(
        _mlp_kernel,
        grid=grid,
        in_specs=x_specs + [
            pl.BlockSpec(W1.shape, lambda i: (0, 0)),
            pl.BlockSpec(b1r.shape, lambda i: (0, 0)),
            pl.BlockSpec(W2.shape, lambda i: (0, 0)),
            pl.BlockSpec(b2r.shape, lambda i: (0, 0)),
            pl.BlockSpec(W3.shape, lambda i: (0, 0)),
        ],
        out_specs=pl.BlockSpec((_NOUT, _BLK), lambda i: (0, i)),
        out_shape=jax.ShapeDtypeStruct((_NOUT, B), jnp.float32),
    )(*([input] * _NSPLIT), W1, b1r, W2, b2r, W3)
    return (out.sum(axis=0) + b3[0]).reshape(B, 1)


# speedup vs baseline: 1.0984x; 1.0984x over previous
import jax
import jax.numpy as jnp
from jax.experimental import pallas as pl
from jax.experimental.pallas import tpu as pltpu

_BLK = 16384
_NSPLIT = 4
_SUB = _BLK // _NSPLIT
_NOUT = 8


def _mlp_kernel(*refs):
    x_refs = refs[:_NSPLIT]
    w1_ref, b1_ref, w2_ref, b2_ref, w3_ref, o_ref = refs[_NSPLIT:]
    w1 = w1_ref[...].astype(jnp.bfloat16)
    w2 = w2_ref[...].astype(jnp.bfloat16)
    b1 = b1_ref[...].astype(jnp.bfloat16)
    b2 = b2_ref[...].astype(jnp.bfloat16)
    w3 = jnp.broadcast_to(w3_ref[...] * 0.125, (_NOUT, 64)).astype(jnp.bfloat16)
    for s, x_ref in enumerate(x_refs):
        x = x_ref[...].astype(jnp.bfloat16)
        h = jax.lax.dot_general(
            x, w1, (((1,), (1,)), ((), ())),
            preferred_element_type=jnp.float32)
        h = jnp.maximum(h.astype(jnp.bfloat16) + b1, jnp.bfloat16(0))
        h = jax.lax.dot_general(
            h, w2, (((1,), (1,)), ((), ())),
            preferred_element_type=jnp.float32)
        h = jnp.maximum(h.astype(jnp.bfloat16) + b2, jnp.bfloat16(0))
        o_ref[:, pl.ds(s * _SUB, _SUB)] = jax.lax.dot_general(
            w3, h, (((1,), (1,)), ((), ())),
            preferred_element_type=jnp.float32)


def kernel(input, W1, b1, W2, b2, W3, b3):
    B, D = input.shape
    grid = (B // _BLK,)
    b1r = b1.reshape(1, 64)
    b2r = b2.reshape(1, 64)
    x_specs = [
        pl.BlockSpec((_SUB, D), lambda i, s=s: (i * _NSPLIT + s, 0))
        for s in range(_NSPLIT)
    ]
    out = pl.pallas_call(
        _mlp_kernel,
        grid=grid,
        in_specs=x_specs + [
            pl.BlockSpec(W1.shape, lambda i: (0, 0)),
            pl.BlockSpec(b1r.shape, lambda i: (0, 0)),
            pl.BlockSpec(W2.shape, lambda i: (0, 0)),
            pl.BlockSpec(b2r.shape, lambda i: (0, 0)),
            pl.BlockSpec(W3.shape, lambda i: (0, 0)),
        ],
        out_specs=pl.BlockSpec((_NOUT, _BLK), lambda i: (0, i)),
        out_shape=jax.ShapeDtypeStruct((_NOUT, B), jnp.float32),
    )(*([input] * _NSPLIT), W1, b1r, W2, b2r, W3)
    return (out.sum(axis=0) + b3[0]).reshape(B, 1)
